# Initial kernel scaffold; baseline (speedup 1.0000x reference)
import jax
import jax.numpy as jnp
from jax.experimental import pallas as pl


def kernel(adj, adj_orig, features, W_base, W_mean, W_nc0, b_nc0, W_nc1, b_nc1):
    raise NotImplementedError("write your pallas kernel here")


# speedup vs baseline: 2.7077x; 2.7077x over previous
import functools

import jax
import jax.numpy as jnp
from jax.experimental import pallas as pl
from jax.experimental.pallas import tpu as pltpu

N = 4096
D = 128
H = 32
Z = 16
CPAD = 128

BM = 512
BK = 512
NI = N // BM
NK = N // BK


def _proj_kernel(f_ref, wb_ref, w0_ref, p1_ref, x3_ref):
    f = f_ref[...]
    p1_ref[...] = jnp.dot(f, wb_ref[...], preferred_element_type=jnp.float32)
    x3_ref[...] = jnp.dot(f, w0_ref[...], preferred_element_type=jnp.float32)


def _gcn1_kernel(adj_ref, p1_ref, wm_ref, out_ref, acc_ref):
    k = pl.program_id(1)

    @pl.when(k == 0)
    def _():
        acc_ref[...] = jnp.zeros_like(acc_ref)

    acc_ref[...] += jnp.dot(adj_ref[...], p1_ref[...],
                            preferred_element_type=jnp.float32)

    @pl.when(k == pl.num_programs(1) - 1)
    def _():
        out_ref[...] = jnp.dot(acc_ref[...], wm_ref[...],
                               preferred_element_type=jnp.float32)


def _gcn2_kernel(adj_ref, p2_ref, out_ref, acc_ref):
    k = pl.program_id(1)

    @pl.when(k == 0)
    def _():
        acc_ref[...] = jnp.zeros_like(acc_ref)

    acc_ref[...] += jnp.dot(adj_ref[...], p2_ref[...],
                            preferred_element_type=jnp.float32)

    @pl.when(k == pl.num_programs(1) - 1)
    def _():
        out_ref[...] = jnp.maximum(acc_ref[...], 0.0)


def _gram_kernel(mi_ref, mtj_ref, out_ref, mmax_ref):
    i = pl.program_id(0)
    j = pl.program_id(1)
    g = jnp.dot(mi_ref[...], mtj_ref[...], preferred_element_type=jnp.float32)
    out_ref[...] = g

    @pl.when((i == 0) & (j == 0))
    def _():
        mmax_ref[0, 0] = -jnp.inf

    mmax_ref[0, 0] = jnp.maximum(mmax_ref[0, 0], jnp.max(g))


def _adj_block(mi, mtk, mmax, i, k):
    g = jnp.dot(mi, mtk, preferred_element_type=jnp.float32)
    r = jnp.round(g / mmax)
    rows = jax.lax.broadcasted_iota(jnp.int32, (BM, BK), 0) + i * BM
    cols = jax.lax.broadcasted_iota(jnp.int32, (BM, BK), 1) + k * BK
    return jnp.where(rows == cols, 1.0, r)


def _deg_kernel(mi_ref, mtk_ref, mmax_ref, d_ref, acc_ref):
    i = pl.program_id(0)
    k = pl.program_id(1)

    @pl.when(k == 0)
    def _():
        acc_ref[...] = jnp.zeros_like(acc_ref)

    a = _adj_block(mi_ref[...], mtk_ref[...], mmax_ref[0, 0], i, k)
    acc_ref[...] += jnp.sum(a, axis=1, keepdims=True)

    @pl.when(k == pl.num_programs(1) - 1)
    def _():
        d_ref[...] = jax.lax.pow(acc_ref[...], -0.5)


def _nc1_kernel(mi_ref, mtk_ref, mmax_ref, di_ref, dk_ref, x3_ref, b0_ref,
                w1_ref, y4_ref, acc_ref):
    i = pl.program_id(0)
    k = pl.program_id(1)

    @pl.when(k == 0)
    def _():
        acc_ref[...] = jnp.zeros_like(acc_ref)

    a = _adj_block(mi_ref[...], mtk_ref[...], mmax_ref[0, 0], i, k)
    y3 = dk_ref[...] * x3_ref[...]
    acc_ref[...] += jnp.dot(a, y3, preferred_element_type=jnp.float32)

    @pl.when(k == pl.num_programs(1) - 1)
    def _():
        h = jnp.maximum(di_ref[...] * acc_ref[...] + b0_ref[...], 0.0)
        y4_ref[...] = di_ref[...] * jnp.dot(h, w1_ref[...],
                                            preferred_element_type=jnp.float32)


def _nc2_kernel(mi_ref, mtk_ref, mmax_ref, di_ref, y4_ref, b1_ref, out_ref,
                acc_ref):
    i = pl.program_id(0)
    k = pl.program_id(1)

    @pl.when(k == 0)
    def _():
        acc_ref[...] = jnp.zeros_like(acc_ref)

    a = _adj_block(mi_ref[...], mtk_ref[...], mmax_ref[0, 0], i, k)
    acc_ref[...] += jnp.dot(a, y4_ref[...], preferred_element_type=jnp.float32)

    @pl.when(k == pl.num_programs(1) - 1)
    def _():
        out_ref[...] = di_ref[...] * acc_ref[...] + b1_ref[...]


@jax.jit
def kernel(adj, adj_orig, features, W_base, W_mean, W_nc0, b_nc0, W_nc1,
           b_nc1):
    del adj_orig
    f32 = jnp.float32

    p1, x3 = pl.pallas_call(
        _proj_kernel,
        grid=(NI,),
        in_specs=[
            pl.BlockSpec((BM, D), lambda i: (i, 0)),
            pl.BlockSpec((D, H), lambda i: (0, 0)),
            pl.BlockSpec((D, H), lambda i: (0, 0)),
        ],
        out_specs=[
            pl.BlockSpec((BM, H), lambda i: (i, 0)),
            pl.BlockSpec((BM, H), lambda i: (i, 0)),
        ],
        out_shape=[
            jax.ShapeDtypeStruct((N, H), f32),
            jax.ShapeDtypeStruct((N, H), f32),
        ],
    )(features, W_base, W_nc0)

    p2 = pl.pallas_call(
        _gcn1_kernel,
        grid=(NI, NK),
        in_specs=[
            pl.BlockSpec((BM, BK), lambda i, k: (i, k)),
            pl.BlockSpec((BK, H), lambda i, k: (k, 0)),
            pl.BlockSpec((H, Z), lambda i, k: (0, 0)),
        ],
        out_specs=pl.BlockSpec((BM, Z), lambda i, k: (i, 0)),
        out_shape=jax.ShapeDtypeStruct((N, Z), f32),
        scratch_shapes=[pltpu.VMEM((BM, H), f32)],
    )(adj, p1, W_mean)

    mean = pl.pallas_call(
        _gcn2_kernel,
        grid=(NI, NK),
        in_specs=[
            pl.BlockSpec((BM, BK), lambda i, k: (i, k)),
            pl.BlockSpec((BK, Z), lambda i, k: (k, 0)),
        ],
        out_specs=pl.BlockSpec((BM, Z), lambda i, k: (i, 0)),
        out_shape=jax.ShapeDtypeStruct((N, Z), f32),
        scratch_shapes=[pltpu.VMEM((BM, Z), f32)],
    )(adj, p2)

    meant = mean.T

    adj_logits, mmax = pl.pallas_call(
        _gram_kernel,
        grid=(NI, NK),
        in_specs=[
            pl.BlockSpec((BM, Z), lambda i, j: (i, 0)),
            pl.BlockSpec((Z, BK), lambda i, j: (0, j)),
        ],
        out_specs=[
            pl.BlockSpec((BM, BK), lambda i, j: (i, j)),
            pl.BlockSpec((1, 1), lambda i, j: (0, 0),
                         memory_space=pltpu.SMEM),
        ],
        out_shape=[
            jax.ShapeDtypeStruct((N, N), f32),
            jax.ShapeDtypeStruct((1, 1), f32),
        ],
    )(mean, meant)

    dvec = pl.pallas_call(
        _deg_kernel,
        grid=(NI, NK),
        in_specs=[
            pl.BlockSpec((BM, Z), lambda i, k: (i, 0)),
            pl.BlockSpec((Z, BK), lambda i, k: (0, k)),
            pl.BlockSpec((1, 1), lambda i, k: (0, 0),
                         memory_space=pltpu.SMEM),
        ],
        out_specs=pl.BlockSpec((BM, 1), lambda i, k: (i, 0)),
        out_shape=jax.ShapeDtypeStruct((N, 1), f32),
        scratch_shapes=[pltpu.VMEM((BM, 1), f32)],
    )(mean, meant, mmax)

    b0 = b_nc0.reshape(1, H)
    w1p = jnp.zeros((H, CPAD), f32).at[:, :b_nc1.shape[0]].set(W_nc1)
    b1p = jnp.zeros((1, CPAD), f32).at[0, :b_nc1.shape[0]].set(b_nc1)

    y4 = pl.pallas_call(
        _nc1_kernel,
        grid=(NI, NK),
        in_specs=[
            pl.BlockSpec((BM, Z), lambda i, k: (i, 0)),
            pl.BlockSpec((Z, BK), lambda i, k: (0, k)),
            pl.BlockSpec((1, 1), lambda i, k: (0, 0),
                         memory_space=pltpu.SMEM),
            pl.BlockSpec((BM, 1), lambda i, k: (i, 0)),
            pl.BlockSpec((BK, 1), lambda i, k: (k, 0)),
            pl.BlockSpec((BK, H), lambda i, k: (k, 0)),
            pl.BlockSpec((1, H), lambda i, k: (0, 0)),
            pl.BlockSpec((H, CPAD), lambda i, k: (0, 0)),
        ],
        out_specs=pl.BlockSpec((BM, CPAD), lambda i, k: (i, 0)),
        out_shape=jax.ShapeDtypeStruct((N, CPAD), f32),
        scratch_shapes=[pltpu.VMEM((BM, H), f32)],
    )(mean, meant, mmax, dvec, dvec, x3, b0, w1p)

    ncp = pl.pallas_call(
        _nc2_kernel,
        grid=(NI, NK),
        in_specs=[
            pl.BlockSpec((BM, Z), lambda i, k: (i, 0)),
            pl.BlockSpec((Z, BK), lambda i, k: (0, k)),
            pl.BlockSpec((1, 1), lambda i, k: (0, 0),
                         memory_space=pltpu.SMEM),
            pl.BlockSpec((BM, 1), lambda i, k: (i, 0)),
            pl.BlockSpec((BK, CPAD), lambda i, k: (k, 0)),
            pl.BlockSpec((1, CPAD), lambda i, k: (0, 0)),
        ],
        out_specs=pl.BlockSpec((BM, CPAD), lambda i, k: (i, 0)),
        out_shape=jax.ShapeDtypeStruct((N, CPAD), f32),
        scratch_shapes=[pltpu.VMEM((BM, CPAD), f32)],
    )(mean, meant, mmax, dvec, y4, b1p)

    nc_logits = ncp[:, :b_nc1.shape[0]]
    return (nc_logits, adj_logits)


# speedup vs baseline: 3.0235x; 1.1166x over previous
import jax
import jax.numpy as jnp
from jax.experimental import pallas as pl
from jax.experimental.pallas import tpu as pltpu

N = 4096
D = 128
H = 32
Z = 16
CPAD = 128

BM = 512
BK = 512
NI = N // BM
NK = N // BK


def _proj_kernel(f_ref, wb_ref, w0_ref, p1_ref, x3_ref):
    f = f_ref[...]
    p1_ref[...] = jnp.dot(f, wb_ref[...], preferred_element_type=jnp.float32)
    x3_ref[...] = jnp.dot(f, w0_ref[...], preferred_element_type=jnp.float32)


def _gcn1_kernel(adj_ref, p1_ref, wm_ref, out_ref, acc_ref):
    k = pl.program_id(1)

    @pl.when(k == 0)
    def _():
        acc_ref[...] = jnp.zeros_like(acc_ref)

    acc_ref[...] += jnp.dot(adj_ref[...], p1_ref[...],
                            preferred_element_type=jnp.float32)

    @pl.when(k == pl.num_programs(1) - 1)
    def _():
        out_ref[...] = jnp.dot(acc_ref[...], wm_ref[...],
                               preferred_element_type=jnp.float32)


def _gcn2_kernel(adj_ref, p2_ref, mean_ref, mmax_ref, acc_ref):
    i = pl.program_id(0)
    k = pl.program_id(1)

    @pl.when(k == 0)
    def _():
        acc_ref[...] = jnp.zeros_like(acc_ref)

    acc_ref[...] += jnp.dot(adj_ref[...], p2_ref[...],
                            preferred_element_type=jnp.float32)

    @pl.when(k == pl.num_programs(1) - 1)
    def _():
        m = jnp.maximum(acc_ref[...], 0.0)
        mean_ref[...] = m
        blk = jnp.max(jnp.sum(m * m, axis=1))

        @pl.when(i == 0)
        def _():
            mmax_ref[0, 0] = blk

        mmax_ref[0, 0] = jnp.maximum(mmax_ref[0, 0], blk)


def _gram_deg_kernel(mi_ref, mtj_ref, mmax_ref, out_ref, d_ref, rdiag_ref,
                     acc_ref, rd_ref):
    i = pl.program_id(0)
    j = pl.program_id(1)

    @pl.when(j == 0)
    def _():
        acc_ref[...] = jnp.zeros_like(acc_ref)

    g = jnp.dot(mi_ref[...], mtj_ref[...], preferred_element_type=jnp.float32)
    out_ref[...] = g
    r = jnp.round(g * (1.0 / mmax_ref[0, 0]))
    acc_ref[...] += jnp.sum(r, axis=1, keepdims=True)

    @pl.when(i == j)
    def _():
        rows = jax.lax.broadcasted_iota(jnp.int32, (BM, BK), 0)
        cols = jax.lax.broadcasted_iota(jnp.int32, (BM, BK), 1)
        rd_ref[...] = jnp.sum(jnp.where(rows == cols, r, 0.0), axis=1,
                              keepdims=True)

    @pl.when(j == pl.num_programs(1) - 1)
    def _():
        rd = rd_ref[...]
        rdiag_ref[...] = rd
        d_ref[...] = jax.lax.pow(acc_ref[...] + (1.0 - rd), -0.5)


def _r_block(mi, mtk, mmax):
    g = jnp.dot(mi, mtk, preferred_element_type=jnp.float32)
    return jnp.round(g * (1.0 / mmax))


def _nc1_kernel(mi_ref, mtk_ref, mmax_ref, di_ref, dk_ref, x3k_ref, x3i_ref,
                rdiag_ref, b0_ref, w1_ref, y4_ref, acc_ref):
    k = pl.program_id(1)

    @pl.when(k == 0)
    def _():
        acc_ref[...] = jnp.zeros_like(acc_ref)

    r = _r_block(mi_ref[...], mtk_ref[...], mmax_ref[0, 0])
    acc_ref[...] += jnp.dot(r, dk_ref[...] * x3k_ref[...],
                            preferred_element_type=jnp.float32)

    @pl.when(k == pl.num_programs(1) - 1)
    def _():
        di = di_ref[...]
        acc = acc_ref[...] + (1.0 - rdiag_ref[...]) * (di * x3i_ref[...])
        h = jnp.maximum(di * acc + b0_ref[...], 0.0)
        y4_ref[...] = di * jnp.dot(h, w1_ref[...],
                                   preferred_element_type=jnp.float32)


def _nc2_kernel(mi_ref, mtk_ref, mmax_ref, di_ref, rdiag_ref, y4k_ref,
                y4i_ref, b1_ref, out_ref, acc_ref):
    k = pl.program_id(1)

    @pl.when(k == 0)
    def _():
        acc_ref[...] = jnp.zeros_like(acc_ref)

    r = _r_block(mi_ref[...], mtk_ref[...], mmax_ref[0, 0])
    acc_ref[...] += jnp.dot(r, y4k_ref[...],
                            preferred_element_type=jnp.float32)

    @pl.when(k == pl.num_programs(1) - 1)
    def _():
        acc = acc_ref[...] + (1.0 - rdiag_ref[...]) * y4i_ref[...]
        out_ref[...] = di_ref[...] * acc + b1_ref[...]


@jax.jit
def kernel(adj, adj_orig, features, W_base, W_mean, W_nc0, b_nc0, W_nc1,
           b_nc1):
    del adj_orig
    f32 = jnp.float32

    p1, x3 = pl.pallas_call(
        _proj_kernel,
        grid=(NI,),
        in_specs=[
            pl.BlockSpec((BM, D), lambda i: (i, 0)),
            pl.BlockSpec((D, H), lambda i: (0, 0)),
            pl.BlockSpec((D, H), lambda i: (0, 0)),
        ],
        out_specs=[
            pl.BlockSpec((BM, H), lambda i: (i, 0)),
            pl.BlockSpec((BM, H), lambda i: (i, 0)),
        ],
        out_shape=[
            jax.ShapeDtypeStruct((N, H), f32),
            jax.ShapeDtypeStruct((N, H), f32),
        ],
    )(features, W_base, W_nc0)

    p2 = pl.pallas_call(
        _gcn1_kernel,
        grid=(NI, NK),
        in_specs=[
            pl.BlockSpec((BM, BK), lambda i, k: (i, k)),
            pl.BlockSpec((BK, H), lambda i, k: (k, 0)),
            pl.BlockSpec((H, Z), lambda i, k: (0, 0)),
        ],
        out_specs=pl.BlockSpec((BM, Z), lambda i, k: (i, 0)),
        out_shape=jax.ShapeDtypeStruct((N, Z), f32),
        scratch_shapes=[pltpu.VMEM((BM, H), f32)],
    )(adj, p1, W_mean)

    mean, mmax = pl.pallas_call(
        _gcn2_kernel,
        grid=(NI, NK),
        in_specs=[
            pl.BlockSpec((BM, BK), lambda i, k: (i, k)),
            pl.BlockSpec((BK, Z), lambda i, k: (k, 0)),
        ],
        out_specs=[
            pl.BlockSpec((BM, Z), lambda i, k: (i, 0)),
            pl.BlockSpec((1, 1), lambda i, k: (0, 0),
                         memory_space=pltpu.SMEM),
        ],
        out_shape=[
            jax.ShapeDtypeStruct((N, Z), f32),
            jax.ShapeDtypeStruct((1, 1), f32),
        ],
        scratch_shapes=[pltpu.VMEM((BM, Z), f32)],
    )(adj, p2)

    meant = mean.T

    adj_logits, dvec, rdiag = pl.pallas_call(
        _gram_deg_kernel,
        grid=(NI, NK),
        in_specs=[
            pl.BlockSpec((BM, Z), lambda i, j: (i, 0)),
            pl.BlockSpec((Z, BK), lambda i, j: (0, j)),
            pl.BlockSpec((1, 1), lambda i, j: (0, 0),
                         memory_space=pltpu.SMEM),
        ],
        out_specs=[
            pl.BlockSpec((BM, BK), lambda i, j: (i, j)),
            pl.BlockSpec((BM, 1), lambda i, j: (i, 0)),
            pl.BlockSpec((BM, 1), lambda i, j: (i, 0)),
        ],
        out_shape=[
            jax.ShapeDtypeStruct((N, N), f32),
            jax.ShapeDtypeStruct((N, 1), f32),
            jax.ShapeDtypeStruct((N, 1), f32),
        ],
        scratch_shapes=[pltpu.VMEM((BM, 1), f32), pltpu.VMEM((BM, 1), f32)],
    )(mean, meant, mmax)

    b0 = b_nc0.reshape(1, H)
    w1p = jnp.zeros((H, CPAD), f32).at[:, :b_nc1.shape[0]].set(W_nc1)
    b1p = jnp.zeros((1, CPAD), f32).at[0, :b_nc1.shape[0]].set(b_nc1)

    y4 = pl.pallas_call(
        _nc1_kernel,
        grid=(NI, NK),
        in_specs=[
            pl.BlockSpec((BM, Z), lambda i, k: (i, 0)),
            pl.BlockSpec((Z, BK), lambda i, k: (0, k)),
            pl.BlockSpec((1, 1), lambda i, k: (0, 0),
                         memory_space=pltpu.SMEM),
            pl.BlockSpec((BM, 1), lambda i, k: (i, 0)),
            pl.BlockSpec((BK, 1), lambda i, k: (k, 0)),
            pl.BlockSpec((BK, H), lambda i, k: (k, 0)),
            pl.BlockSpec((BM, H), lambda i, k: (i, 0)),
            pl.BlockSpec((BM, 1), lambda i, k: (i, 0)),
            pl.BlockSpec((1, H), lambda i, k: (0, 0)),
            pl.BlockSpec((H, CPAD), lambda i, k: (0, 0)),
        ],
        out_specs=pl.BlockSpec((BM, CPAD), lambda i, k: (i, 0)),
        out_shape=jax.ShapeDtypeStruct((N, CPAD), f32),
        scratch_shapes=[pltpu.VMEM((BM, H), f32)],
    )(mean, meant, mmax, dvec, dvec, x3, x3, rdiag, b0, w1p)

    ncp = pl.pallas_call(
        _nc2_kernel,
        grid=(NI, NK),
        in_specs=[
            pl.BlockSpec((BM, Z), lambda i, k: (i, 0)),
            pl.BlockSpec((Z, BK), lambda i, k: (0, k)),
            pl.BlockSpec((1, 1), lambda i, k: (0, 0),
                         memory_space=pltpu.SMEM),
            pl.BlockSpec((BM, 1), lambda i, k: (i, 0)),
            pl.BlockSpec((BM, 1), lambda i, k: (i, 0)),
            pl.BlockSpec((BK, CPAD), lambda i, k: (k, 0)),
            pl.BlockSpec((BM, CPAD), lambda i, k: (i, 0)),
            pl.BlockSpec((1, CPAD), lambda i, k: (0, 0)),
        ],
        out_specs=pl.BlockSpec((BM, CPAD), lambda i, k: (i, 0)),
        out_shape=jax.ShapeDtypeStruct((N, CPAD), f32),
        scratch_shapes=[pltpu.VMEM((BM, CPAD), f32)],
    )(mean, meant, mmax, dvec, rdiag, y4, y4, b1p)

    nc_logits = ncp[:, :b_nc1.shape[0]]
    return (nc_logits, adj_logits)


# speedup vs baseline: 4.2870x; 1.4179x over previous
import jax
import jax.numpy as jnp
from jax.experimental import pallas as pl
from jax.experimental.pallas import tpu as pltpu

N = 4096
D = 128
H = 32
Z = 16
CPAD = 128

BM = 512
BK = 1024
NI = N // BM
NK = N // BK


def _proj_kernel(f_ref, wb_ref, w0_ref, p1_ref, x3_ref):
    f = f_ref[...]
    p1_ref[...] = jnp.dot(f, wb_ref[...], preferred_element_type=jnp.float32)
    x3_ref[...] = jnp.dot(f, w0_ref[...], preferred_element_type=jnp.float32)


def _gcn1_kernel(adj_ref, p1_ref, wm_ref, out_ref, acc_ref):
    k = pl.program_id(1)

    @pl.when(k == 0)
    def _():
        acc_ref[...] = jnp.zeros_like(acc_ref)

    acc_ref[...] += jnp.dot(adj_ref[...], p1_ref[...],
                            preferred_element_type=jnp.float32)

    @pl.when(k == pl.num_programs(1) - 1)
    def _():
        out_ref[...] = jnp.dot(acc_ref[...], wm_ref[...],
                               preferred_element_type=jnp.float32)


def _gcn2_kernel(adj_ref, p2_ref, mean_ref, mmax_ref, acc_ref):
    i = pl.program_id(0)
    k = pl.program_id(1)

    @pl.when(k == 0)
    def _():
        acc_ref[...] = jnp.zeros_like(acc_ref)

    acc_ref[...] += jnp.dot(adj_ref[...], p2_ref[...],
                            preferred_element_type=jnp.float32)

    @pl.when(k == pl.num_programs(1) - 1)
    def _():
        m = jnp.maximum(acc_ref[...], 0.0)
        mean_ref[...] = m
        blk = jnp.max(jnp.sum(m * m, axis=1))

        @pl.when(i == 0)
        def _():
            mmax_ref[0, 0] = blk

        mmax_ref[0, 0] = jnp.maximum(mmax_ref[0, 0], blk)


def _gram_deg_kernel(mi_ref, mtj_ref, mmax_ref, out_ref, d_ref, rdiag_ref,
                     acc_ref, rd_ref):
    i = pl.program_id(0)
    j = pl.program_id(1)

    @pl.when(j == 0)
    def _():
        acc_ref[...] = jnp.zeros_like(acc_ref)

    g = jnp.dot(mi_ref[...], mtj_ref[...], preferred_element_type=jnp.float32)
    out_ref[...] = g
    r = jnp.round(g * (1.0 / mmax_ref[0, 0]))
    acc_ref[...] += jnp.sum(r, axis=1, keepdims=True)

    @pl.when(j == (i * BM) // BK)
    def _():
        rows = jax.lax.broadcasted_iota(jnp.int32, (BM, BK), 0) + i * BM
        cols = jax.lax.broadcasted_iota(jnp.int32, (BM, BK), 1) + j * BK
        rd_ref[...] = jnp.sum(jnp.where(rows == cols, r, 0.0), axis=1,
                              keepdims=True)

    @pl.when(j == pl.num_programs(1) - 1)
    def _():
        rd = rd_ref[...]
        rdiag_ref[...] = rd
        d_ref[...] = jax.lax.pow(acc_ref[...] + (1.0 - rd), -0.5)


def _r_block(mi, mtk, mmax):
    g = jnp.dot(mi, mtk, preferred_element_type=jnp.float32)
    return jnp.round(g * (1.0 / mmax))


def _nc1_kernel(mi_ref, mtk_ref, mmax_ref, di_ref, dk_ref, x3k_ref, x3i_ref,
                rdiag_ref, b0_ref, w1_ref, y4_ref, acc_ref):
    k = pl.program_id(1)

    @pl.when(k == 0)
    def _():
        acc_ref[...] = jnp.zeros_like(acc_ref)

    r = _r_block(mi_ref[...], mtk_ref[...], mmax_ref[0, 0])
    acc_ref[...] += jnp.dot(r, dk_ref[...] * x3k_ref[...],
                            preferred_element_type=jnp.float32)

    @pl.when(k == pl.num_programs(1) - 1)
    def _():
        di = di_ref[...]
        acc = acc_ref[...] + (1.0 - rdiag_ref[...]) * (di * x3i_ref[...])
        h = jnp.maximum(di * acc + b0_ref[...], 0.0)
        y4_ref[...] = di * jnp.dot(h, w1_ref[...],
                                   preferred_element_type=jnp.float32)


def _nc2_kernel(mi_ref, mtk_ref, mmax_ref, di_ref, rdiag_ref, y4k_ref,
                y4i_ref, b1_ref, out_ref, acc_ref):
    k = pl.program_id(1)

    @pl.when(k == 0)
    def _():
        acc_ref[...] = jnp.zeros_like(acc_ref)

    r = _r_block(mi_ref[...], mtk_ref[...], mmax_ref[0, 0])
    acc_ref[...] += jnp.dot(r, y4k_ref[...],
                            preferred_element_type=jnp.float32)

    @pl.when(k == pl.num_programs(1) - 1)
    def _():
        acc = acc_ref[...] + (1.0 - rdiag_ref[...]) * y4i_ref[...]
        out_ref[...] = di_ref[...] * acc + b1_ref[...]


@jax.jit
def kernel(adj, adj_orig, features, W_base, W_mean, W_nc0, b_nc0, W_nc1,
           b_nc1):
    del adj_orig
    f32 = jnp.float32

    p1, x3 = pl.pallas_call(
        _proj_kernel,
        grid=(NI,),
        in_specs=[
            pl.BlockSpec((BM, D), lambda i: (i, 0)),
            pl.BlockSpec((D, H), lambda i: (0, 0)),
            pl.BlockSpec((D, H), lambda i: (0, 0)),
        ],
        out_specs=[
            pl.BlockSpec((BM, H), lambda i: (i, 0)),
            pl.BlockSpec((BM, H), lambda i: (i, 0)),
        ],
        out_shape=[
            jax.ShapeDtypeStruct((N, H), f32),
            jax.ShapeDtypeStruct((N, H), f32),
        ],
    )(features, W_base, W_nc0)

    p2 = pl.pallas_call(
        _gcn1_kernel,
        grid=(NI, NK),
        in_specs=[
            pl.BlockSpec((BM, BK), lambda i, k: (i, k)),
            pl.BlockSpec((BK, H), lambda i, k: (k, 0)),
            pl.BlockSpec((H, Z), lambda i, k: (0, 0)),
        ],
        out_specs=pl.BlockSpec((BM, Z), lambda i, k: (i, 0)),
        out_shape=jax.ShapeDtypeStruct((N, Z), f32),
        scratch_shapes=[pltpu.VMEM((BM, H), f32)],
    )(adj, p1, W_mean)

    mean, mmax = pl.pallas_call(
        _gcn2_kernel,
        grid=(NI, NK),
        in_specs=[
            pl.BlockSpec((BM, BK), lambda i, k: (i, k)),
            pl.BlockSpec((BK, Z), lambda i, k: (k, 0)),
        ],
        out_specs=[
            pl.BlockSpec((BM, Z), lambda i, k: (i, 0)),
            pl.BlockSpec((1, 1), lambda i, k: (0, 0),
                         memory_space=pltpu.SMEM),
        ],
        out_shape=[
            jax.ShapeDtypeStruct((N, Z), f32),
            jax.ShapeDtypeStruct((1, 1), f32),
        ],
        scratch_shapes=[pltpu.VMEM((BM, Z), f32)],
    )(adj, p2)

    meant = mean.T

    adj_logits, dvec, rdiag = pl.pallas_call(
        _gram_deg_kernel,
        grid=(NI, NK),
        in_specs=[
            pl.BlockSpec((BM, Z), lambda i, j: (i, 0)),
            pl.BlockSpec((Z, BK), lambda i, j: (0, j)),
            pl.BlockSpec((1, 1), lambda i, j: (0, 0),
                         memory_space=pltpu.SMEM),
        ],
        out_specs=[
            pl.BlockSpec((BM, BK), lambda i, j: (i, j)),
            pl.BlockSpec((BM, 1), lambda i, j: (i, 0)),
            pl.BlockSpec((BM, 1), lambda i, j: (i, 0)),
        ],
        out_shape=[
            jax.ShapeDtypeStruct((N, N), f32),
            jax.ShapeDtypeStruct((N, 1), f32),
            jax.ShapeDtypeStruct((N, 1), f32),
        ],
        scratch_shapes=[pltpu.VMEM((BM, 1), f32), pltpu.VMEM((BM, 1), f32)],
    )(mean, meant, mmax)

    b0 = b_nc0.reshape(1, H)
    w1p = jnp.zeros((H, CPAD), f32).at[:, :b_nc1.shape[0]].set(W_nc1)
    b1p = jnp.zeros((1, CPAD), f32).at[0, :b_nc1.shape[0]].set(b_nc1)

    y4 = pl.pallas_call(
        _nc1_kernel,
        grid=(NI, NK),
        in_specs=[
            pl.BlockSpec((BM, Z), lambda i, k: (i, 0)),
            pl.BlockSpec((Z, BK), lambda i, k: (0, k)),
            pl.BlockSpec((1, 1), lambda i, k: (0, 0),
                         memory_space=pltpu.SMEM),
            pl.BlockSpec((BM, 1), lambda i, k: (i, 0)),
            pl.BlockSpec((BK, 1), lambda i, k: (k, 0)),
            pl.BlockSpec((BK, H), lambda i, k: (k, 0)),
            pl.BlockSpec((BM, H), lambda i, k: (i, 0)),
            pl.BlockSpec((BM, 1), lambda i, k: (i, 0)),
            pl.BlockSpec((1, H), lambda i, k: (0, 0)),
            pl.BlockSpec((H, CPAD), lambda i, k: (0, 0)),
        ],
        out_specs=pl.BlockSpec((BM, CPAD), lambda i, k: (i, 0)),
        out_shape=jax.ShapeDtypeStruct((N, CPAD), f32),
        scratch_shapes=[pltpu.VMEM((BM, H), f32)],
    )(mean, meant, mmax, dvec, dvec, x3, x3, rdiag, b0, w1p)

    ncp = pl.pallas_call(
        _nc2_kernel,
        grid=(NI, NK),
        in_specs=[
            pl.BlockSpec((BM, Z), lambda i, k: (i, 0)),
            pl.BlockSpec((Z, BK), lambda i, k: (0, k)),
            pl.BlockSpec((1, 1), lambda i, k: (0, 0),
                         memory_space=pltpu.SMEM),
            pl.BlockSpec((BM, 1), lambda i, k: (i, 0)),
            pl.BlockSpec((BM, 1), lambda i, k: (i, 0)),
            pl.BlockSpec((BK, CPAD), lambda i, k: (k, 0)),
            pl.BlockSpec((BM, CPAD), lambda i, k: (i, 0)),
            pl.BlockSpec((1, CPAD), lambda i, k: (0, 0)),
        ],
        out_specs=pl.BlockSpec((BM, CPAD), lambda i, k: (i, 0)),
        out_shape=jax.ShapeDtypeStruct((N, CPAD), f32),
        scratch_shapes=[pltpu.VMEM((BM, CPAD), f32)],
    )(mean, meant, mmax, dvec, rdiag, y4, y4, b1p)

    nc_logits = ncp[:, :b_nc1.shape[0]]
    return (nc_logits, adj_logits)


# speedup vs baseline: 4.9093x; 1.1452x over previous
import jax
import jax.numpy as jnp
from jax.experimental import pallas as pl
from jax.experimental.pallas import tpu as pltpu

N = 4096
D = 128
H = 32
Z = 16
CPAD = 128

BM = 512
BK = 1024
NI = N // BM
NK = N // BK

P0 = NI
P1 = P0 + NI * NK
P2 = P1 + NI * NK
P3 = P2 + NI * NK
P4 = P3 + NI * NK
P5 = P4 + NI * NK


def _phase_ik(s, lo):
    t = s - lo
    return t // NK, t % NK


def _mega_kernel(adj_ref, f_ref, wb_ref, w0_ref, wm_ref, b0_ref, w1_ref,
                 b1_ref, ncp_ref, logits_ref,
                 p1_s, x3_s, p2_s, mean_s, y4_s, dvec_s, rdiag_s,
                 acc32, acc16, acc128, dacc, rd_s, mmax_s):
    s = pl.program_id(0)

    @pl.when(s < P0)
    def _():
        f = f_ref[...]
        p1_s[pl.ds(s * BM, BM), :] = jnp.dot(
            f, wb_ref[...], preferred_element_type=jnp.float32)
        x3_s[pl.ds(s * BM, BM), :] = jnp.dot(
            f, w0_ref[...], preferred_element_type=jnp.float32)

    @pl.when((s >= P0) & (s < P1))
    def _():
        i, k = _phase_ik(s, P0)

        @pl.when(k == 0)
        def _():
            acc32[...] = jnp.zeros_like(acc32)

        acc32[...] += jnp.dot(adj_ref[...], p1_s[pl.ds(k * BK, BK), :],
                              preferred_element_type=jnp.float32)

        @pl.when(k == NK - 1)
        def _():
            p2_s[pl.ds(i * BM, BM), :] = jnp.dot(
                acc32[...], wm_ref[...], preferred_element_type=jnp.float32)

    @pl.when((s >= P1) & (s < P2))
    def _():
        i, k = _phase_ik(s, P1)

        @pl.when(k == 0)
        def _():
            acc16[...] = jnp.zeros_like(acc16)

        acc16[...] += jnp.dot(adj_ref[...], p2_s[pl.ds(k * BK, BK), :],
                              preferred_element_type=jnp.float32)

        @pl.when(k == NK - 1)
        def _():
            m = jnp.maximum(acc16[...], 0.0)
            mean_s[pl.ds(i * BM, BM), :] = m
            blk = jnp.max(jnp.sum(m * m, axis=1))

            @pl.when(i == 0)
            def _():
                mmax_s[0, 0] = blk

            mmax_s[0, 0] = jnp.maximum(mmax_s[0, 0], blk)

    @pl.when((s >= P2) & (s < P3))
    def _():
        i, j = _phase_ik(s, P2)

        @pl.when(j == 0)
        def _():
            dacc[...] = jnp.zeros_like(dacc)

        mi = mean_s[pl.ds(i * BM, BM), :]
        mj = mean_s[pl.ds(j * BK, BK), :]
        g = jax.lax.dot_general(mi, mj, (((1,), (1,)), ((), ())),
                                preferred_element_type=jnp.float32)
        logits_ref[...] = g
        r = jnp.round(g * (1.0 / mmax_s[0, 0]))
        dacc[...] += jnp.sum(r, axis=1, keepdims=True)

        @pl.when(j == (i * BM) // BK)
        def _():
            rows = jax.lax.broadcasted_iota(jnp.int32, (BM, BK), 0) + i * BM
            cols = jax.lax.broadcasted_iota(jnp.int32, (BM, BK), 1) + j * BK
            rd_s[...] = jnp.sum(jnp.where(rows == cols, r, 0.0), axis=1,
                                keepdims=True)

        @pl.when(j == NK - 1)
        def _():
            rd = rd_s[...]
            rdiag_s[pl.ds(i * BM, BM), :] = rd
            dvec_s[pl.ds(i * BM, BM), :] = jax.lax.pow(
                dacc[...] + (1.0 - rd), -0.5)

    @pl.when((s >= P3) & (s < P4))
    def _():
        i, k = _phase_ik(s, P3)

        @pl.when(k == 0)
        def _():
            acc32[...] = jnp.zeros_like(acc32)

        mi = mean_s[pl.ds(i * BM, BM), :]
        mk = mean_s[pl.ds(k * BK, BK), :]
        g = jax.lax.dot_general(mi, mk, (((1,), (1,)), ((), ())),
                                preferred_element_type=jnp.float32)
        r = jnp.round(g * (1.0 / mmax_s[0, 0]))
        y3 = dvec_s[pl.ds(k * BK, BK), :] * x3_s[pl.ds(k * BK, BK), :]
        acc32[...] += jnp.dot(r, y3, preferred_element_type=jnp.float32)

        @pl.when(k == NK - 1)
        def _():
            di = dvec_s[pl.ds(i * BM, BM), :]
            fix = (1.0 - rdiag_s[pl.ds(i * BM, BM), :])
            acc = acc32[...] + fix * (di * x3_s[pl.ds(i * BM, BM), :])
            h = jnp.maximum(di * acc + b0_ref[...], 0.0)
            y4_s[pl.ds(i * BM, BM), :] = di * jnp.dot(
                h, w1_ref[...], preferred_element_type=jnp.float32)

    @pl.when(s >= P4)
    def _():
        i, k = _phase_ik(s, P4)

        @pl.when(k == 0)
        def _():
            acc128[...] = jnp.zeros_like(acc128)

        mi = mean_s[pl.ds(i * BM, BM), :]
        mk = mean_s[pl.ds(k * BK, BK), :]
        g = jax.lax.dot_general(mi, mk, (((1,), (1,)), ((), ())),
                                preferred_element_type=jnp.float32)
        r = jnp.round(g * (1.0 / mmax_s[0, 0]))
        acc128[...] += jnp.dot(r, y4_s[pl.ds(k * BK, BK), :],
                               preferred_element_type=jnp.float32)

        @pl.when(k == NK - 1)
        def _():
            di = dvec_s[pl.ds(i * BM, BM), :]
            fix = (1.0 - rdiag_s[pl.ds(i * BM, BM), :])
            acc = acc128[...] + fix * y4_s[pl.ds(i * BM, BM), :]
            ncp_ref[...] = di * acc + b1_ref[...]


def _clampi(x, hi):
    return jnp.minimum(x, hi)


def _adj_idx(s):
    in1 = (s >= P0) & (s < P1)
    in2 = (s >= P1) & (s < P2)
    t = jnp.where(in1, s - P0, jnp.where(in2, s - P1, (NI * NK) - 1))
    return t // NK, t % NK


def _feat_idx(s):
    return _clampi(s, NI - 1), 0


def _logits_idx(s):
    t = jnp.clip(s - P2, 0, NI * NK - 1)
    return t // NK, t % NK


def _ncp_idx(s):
    t = jnp.clip(s - P4, 0, NI * NK - 1)
    return t // NK, 0


@jax.jit
def kernel(adj, adj_orig, features, W_base, W_mean, W_nc0, b_nc0, W_nc1,
                b_nc1):
    del adj_orig
    f32 = jnp.float32
    b0 = b_nc0.reshape(1, H)
    w1p = jnp.zeros((H, CPAD), f32).at[:, :b_nc1.shape[0]].set(W_nc1)
    b1p = jnp.zeros((1, CPAD), f32).at[0, :b_nc1.shape[0]].set(b_nc1)
    const = lambda a, b: (lambda s: (a, b))

    ncp, adj_logits = pl.pallas_call(
        _mega_kernel,
        grid=(P5,),
        in_specs=[
            pl.BlockSpec((BM, BK), _adj_idx),
            pl.BlockSpec((BM, D), _feat_idx),
            pl.BlockSpec((D, H), const(0, 0)),
            pl.BlockSpec((D, H), const(0, 0)),
            pl.BlockSpec((H, Z), const(0, 0)),
            pl.BlockSpec((1, H), const(0, 0)),
            pl.BlockSpec((H, CPAD), const(0, 0)),
            pl.BlockSpec((1, CPAD), const(0, 0)),
        ],
        out_specs=[
            pl.BlockSpec((BM, CPAD), _ncp_idx),
            pl.BlockSpec((BM, BK), _logits_idx),
        ],
        out_shape=[
            jax.ShapeDtypeStruct((N, CPAD), f32),
            jax.ShapeDtypeStruct((N, N), f32),
        ],
        scratch_shapes=[
            pltpu.VMEM((N, H), f32),
            pltpu.VMEM((N, H), f32),
            pltpu.VMEM((N, Z), f32),
            pltpu.VMEM((N, Z), f32),
            pltpu.VMEM((N, CPAD), f32),
            pltpu.VMEM((N, 1), f32),
            pltpu.VMEM((N, 1), f32),
            pltpu.VMEM((BM, H), f32),
            pltpu.VMEM((BM, Z), f32),
            pltpu.VMEM((BM, CPAD), f32),
            pltpu.VMEM((BM, 1), f32),
            pltpu.VMEM((BM, 1), f32),
            pltpu.SMEM((1, 1), f32),
        ],
    )(adj, features, W_base, W_nc0, W_mean, b0, w1p, b1p)

    return (ncp[:, :b_nc1.shape[0]], adj_logits)


# speedup vs baseline: 5.7919x; 1.1798x over previous
import jax
import jax.numpy as jnp
from jax.experimental import pallas as pl
from jax.experimental.pallas import tpu as pltpu

N = 4096
D = 128
H = 32
Z = 16
CPAD = 128

BM = 512
BK = 2048
NI = N // BM
NK = N // BK

P0 = NI
P1 = P0 + NI * NK
P2 = P1 + NI * NK
P3 = P2 + NI * NK
P4 = P3 + NI * NK
P5 = P4 + NI * NK


def _phase_ik(s, lo):
    t = s - lo
    return t // NK, t % NK


def _mega_kernel(adj_ref, f_ref, wb_ref, w0_ref, wm_ref, b0_ref, w1_ref,
                 b1_ref, ncp_ref, logits_ref,
                 p1_s, x3_s, p2_s, mean_s, y4_s, dvec_s, rdiag_s,
                 acc32, acc16, acc128, dacc, rd_s, mmax_s):
    s = pl.program_id(0)

    @pl.when(s < P0)
    def _():
        f = f_ref[...]
        p1_s[pl.ds(s * BM, BM), :] = jnp.dot(
            f, wb_ref[...], preferred_element_type=jnp.float32)
        x3_s[pl.ds(s * BM, BM), :] = jnp.dot(
            f, w0_ref[...], preferred_element_type=jnp.float32)

    @pl.when((s >= P0) & (s < P1))
    def _():
        i, k = _phase_ik(s, P0)

        @pl.when(k == 0)
        def _():
            acc32[...] = jnp.zeros_like(acc32)

        acc32[...] += jnp.dot(adj_ref[...], p1_s[pl.ds(k * BK, BK), :],
                              preferred_element_type=jnp.float32)

        @pl.when(k == NK - 1)
        def _():
            p2_s[pl.ds(i * BM, BM), :] = jnp.dot(
                acc32[...], wm_ref[...], preferred_element_type=jnp.float32)

    @pl.when((s >= P1) & (s < P2))
    def _():
        i, k = _phase_ik(s, P1)

        @pl.when(k == 0)
        def _():
            acc16[...] = jnp.zeros_like(acc16)

        acc16[...] += jnp.dot(adj_ref[...], p2_s[pl.ds(k * BK, BK), :],
                              preferred_element_type=jnp.float32)

        @pl.when(k == NK - 1)
        def _():
            m = jnp.maximum(acc16[...], 0.0)
            mean_s[pl.ds(i * BM, BM), :] = m
            blk = jnp.max(jnp.sum(m * m, axis=1))

            @pl.when(i == 0)
            def _():
                mmax_s[0, 0] = blk

            mmax_s[0, 0] = jnp.maximum(mmax_s[0, 0], blk)

    @pl.when((s >= P2) & (s < P3))
    def _():
        i, j = _phase_ik(s, P2)

        @pl.when(j == 0)
        def _():
            dacc[...] = jnp.zeros_like(dacc)

        mi = mean_s[pl.ds(i * BM, BM), :]
        mj = mean_s[pl.ds(j * BK, BK), :]
        g = jax.lax.dot_general(mi, mj, (((1,), (1,)), ((), ())),
                                preferred_element_type=jnp.float32)
        logits_ref[...] = g
        r = jnp.round(g * (1.0 / mmax_s[0, 0]))
        dacc[...] += jnp.dot(r, jnp.ones((BK, 1), jnp.float32),
                             preferred_element_type=jnp.float32)

        @pl.when(j == (i * BM) // BK)
        def _():
            rows = jax.lax.broadcasted_iota(jnp.int32, (BM, BK), 0) + i * BM
            cols = jax.lax.broadcasted_iota(jnp.int32, (BM, BK), 1) + j * BK
            rd_s[...] = jnp.dot(jnp.where(rows == cols, r, 0.0),
                                jnp.ones((BK, 1), jnp.float32),
                                preferred_element_type=jnp.float32)

        @pl.when(j == NK - 1)
        def _():
            rd = rd_s[...]
            rdiag_s[pl.ds(i * BM, BM), :] = rd
            dvec_s[pl.ds(i * BM, BM), :] = jax.lax.pow(
                dacc[...] + (1.0 - rd), -0.5)

    @pl.when((s >= P3) & (s < P4))
    def _():
        i, k = _phase_ik(s, P3)

        @pl.when(k == 0)
        def _():
            acc32[...] = jnp.zeros_like(acc32)

        mi = mean_s[pl.ds(i * BM, BM), :]
        mk = mean_s[pl.ds(k * BK, BK), :]
        g = jax.lax.dot_general(mi, mk, (((1,), (1,)), ((), ())),
                                preferred_element_type=jnp.float32)
        r = jnp.round(g * (1.0 / mmax_s[0, 0]))
        y3 = dvec_s[pl.ds(k * BK, BK), :] * x3_s[pl.ds(k * BK, BK), :]
        acc32[...] += jnp.dot(r, y3, preferred_element_type=jnp.float32)

        @pl.when(k == NK - 1)
        def _():
            di = dvec_s[pl.ds(i * BM, BM), :]
            fix = (1.0 - rdiag_s[pl.ds(i * BM, BM), :])
            acc = acc32[...] + fix * (di * x3_s[pl.ds(i * BM, BM), :])
            h = jnp.maximum(di * acc + b0_ref[...], 0.0)
            y4_s[pl.ds(i * BM, BM), :] = di * jnp.dot(
                h, w1_ref[...], preferred_element_type=jnp.float32)

    @pl.when(s >= P4)
    def _():
        i, k = _phase_ik(s, P4)

        @pl.when(k == 0)
        def _():
            acc128[...] = jnp.zeros_like(acc128)

        mi = mean_s[pl.ds(i * BM, BM), :]
        mk = mean_s[pl.ds(k * BK, BK), :]
        g = jax.lax.dot_general(mi, mk, (((1,), (1,)), ((), ())),
                                preferred_element_type=jnp.float32)
        r = jnp.round(g * (1.0 / mmax_s[0, 0]))
        acc128[...] += jnp.dot(r, y4_s[pl.ds(k * BK, BK), :],
                               preferred_element_type=jnp.float32)

        @pl.when(k == NK - 1)
        def _():
            di = dvec_s[pl.ds(i * BM, BM), :]
            fix = (1.0 - rdiag_s[pl.ds(i * BM, BM), :])
            acc = acc128[...] + fix * y4_s[pl.ds(i * BM, BM), :]
            ncp_ref[...] = di * acc + b1_ref[...]


def _clampi(x, hi):
    return jnp.minimum(x, hi)


def _adj_idx(s):
    in1 = (s >= P0) & (s < P1)
    in2 = (s >= P1) & (s < P2)
    t = jnp.where(in1, s - P0, jnp.where(in2, s - P1, (NI * NK) - 1))
    return t // NK, t % NK


def _feat_idx(s):
    return _clampi(s, NI - 1), 0


def _logits_idx(s):
    t = jnp.clip(s - P2, 0, NI * NK - 1)
    return t // NK, t % NK


def _ncp_idx(s):
    t = jnp.clip(s - P4, 0, NI * NK - 1)
    return t // NK, 0


@jax.jit
def kernel(adj, adj_orig, features, W_base, W_mean, W_nc0, b_nc0, W_nc1,
                b_nc1):
    del adj_orig
    f32 = jnp.float32
    b0 = b_nc0.reshape(1, H)
    w1p = jnp.zeros((H, CPAD), f32).at[:, :b_nc1.shape[0]].set(W_nc1)
    b1p = jnp.zeros((1, CPAD), f32).at[0, :b_nc1.shape[0]].set(b_nc1)
    const = lambda a, b: (lambda s: (a, b))

    ncp, adj_logits = pl.pallas_call(
        _mega_kernel,
        grid=(P5,),
        in_specs=[
            pl.BlockSpec((BM, BK), _adj_idx),
            pl.BlockSpec((BM, D), _feat_idx),
            pl.BlockSpec((D, H), const(0, 0)),
            pl.BlockSpec((D, H), const(0, 0)),
            pl.BlockSpec((H, Z), const(0, 0)),
            pl.BlockSpec((1, H), const(0, 0)),
            pl.BlockSpec((H, CPAD), const(0, 0)),
            pl.BlockSpec((1, CPAD), const(0, 0)),
        ],
        out_specs=[
            pl.BlockSpec((BM, CPAD), _ncp_idx),
            pl.BlockSpec((BM, BK), _logits_idx),
        ],
        out_shape=[
            jax.ShapeDtypeStruct((N, CPAD), f32),
            jax.ShapeDtypeStruct((N, N), f32),
        ],
        scratch_shapes=[
            pltpu.VMEM((N, H), f32),
            pltpu.VMEM((N, H), f32),
            pltpu.VMEM((N, Z), f32),
            pltpu.VMEM((N, Z), f32),
            pltpu.VMEM((N, CPAD), f32),
            pltpu.VMEM((N, 1), f32),
            pltpu.VMEM((N, 1), f32),
            pltpu.VMEM((BM, H), f32),
            pltpu.VMEM((BM, Z), f32),
            pltpu.VMEM((BM, CPAD), f32),
            pltpu.VMEM((BM, 1), f32),
            pltpu.VMEM((BM, 1), f32),
            pltpu.SMEM((1, 1), f32),
        ],
    )(adj, features, W_base, W_nc0, W_mean, b0, w1p, b1p)

    return (ncp[:, :b_nc1.shape[0]], adj_logits)
